# baseline (speedup 1.0000x reference)
import functools

import jax
import jax.numpy as jnp
from jax import lax
from jax.experimental import pallas as pl
from jax.experimental.pallas import tpu as pltpu
from jax.experimental.pallas import tpu_sc as plsc

_NC, _NS = 2, 16
_NW = _NC * _NS
_B = 16384
_D = 32
_BPW = _B // _NW
_CHUNK = 128
_NCH = _BPW // _CHUNK


def _sc_body(uid, iid, ub_t, ib_t, gu_t, gi_t, mut_t, mit_t,
             gmf_o, mu_o, mi_o, bs_o,
             idx_u, idx_i, u_v, i_v, mu_v, mi_v, ub_v, ib_v, sem):
    wid = lax.axis_index("s") * _NC + lax.axis_index("c")
    base = wid * _BPW

    for c in range(_NCH):
        sl = pl.ds(base + c * _CHUNK, _CHUNK)
        pltpu.sync_copy(uid.at[sl], idx_u.at[c])
        pltpu.sync_copy(iid.at[sl], idx_i.at[c])

    handles = []
    for c in range(_NCH):
        handles.append(pltpu.async_copy(gu_t.at[idx_u.at[c]], u_v.at[c], sem))
        handles.append(pltpu.async_copy(gi_t.at[idx_i.at[c]], i_v.at[c], sem))
        handles.append(pltpu.async_copy(mut_t.at[idx_u.at[c]], mu_v.at[c], sem))
        handles.append(pltpu.async_copy(mit_t.at[idx_i.at[c]], mi_v.at[c], sem))
        handles.append(pltpu.async_copy(ub_t.at[idx_u.at[c]], ub_v.at[c], sem))
        handles.append(pltpu.async_copy(ib_t.at[idx_i.at[c]], ib_v.at[c], sem))
    for h in handles:
        h.wait()

    for c in range(_NCH):
        def gbody(r, _, c=c):
            for half in range(_D // 16):
                sl = pl.ds(half * 16, 16)
                u_v[c, r, sl] = u_v[c, r, sl] * i_v[c, r, sl]
            return 0
        lax.fori_loop(0, _CHUNK, gbody, 0, unroll=4)

        def bbody(k, _, c=c):
            sl = pl.ds(k * 16, 16)
            ub_v[c, sl] = ub_v[c, sl] + ib_v[c, sl]
            return 0
        lax.fori_loop(0, _CHUNK // 16, bbody, 0, unroll=4)

    wh = []
    for c in range(_NCH):
        sl = pl.ds(base + c * _CHUNK, _CHUNK)
        wh.append(pltpu.async_copy(u_v.at[c], gmf_o.at[sl], sem))
        wh.append(pltpu.async_copy(mu_v.at[c], mu_o.at[sl], sem))
        wh.append(pltpu.async_copy(mi_v.at[c], mi_o.at[sl], sem))
        wh.append(pltpu.async_copy(ub_v.at[c], bs_o.at[sl], sem))
    for h in wh:
        h.wait()


@functools.cache
def _make_sc_gather():
    return pl.kernel(
        _sc_body,
        out_type=[
            jax.ShapeDtypeStruct((_B, _D), jnp.float32),
            jax.ShapeDtypeStruct((_B, _D), jnp.float32),
            jax.ShapeDtypeStruct((_B, _D), jnp.float32),
            jax.ShapeDtypeStruct((_B,), jnp.float32),
        ],
        mesh=plsc.VectorSubcoreMesh(
            core_axis_name="c", subcore_axis_name="s",
            num_cores=_NC, num_subcores=_NS),
        scratch_types=[
            pltpu.VMEM((_NCH, _CHUNK), jnp.int32),
            pltpu.VMEM((_NCH, _CHUNK), jnp.int32),
            pltpu.VMEM((_NCH, _CHUNK, _D), jnp.float32),
            pltpu.VMEM((_NCH, _CHUNK, _D), jnp.float32),
            pltpu.VMEM((_NCH, _CHUNK, _D), jnp.float32),
            pltpu.VMEM((_NCH, _CHUNK, _D), jnp.float32),
            pltpu.VMEM((_NCH, _CHUNK), jnp.float32),
            pltpu.VMEM((_NCH, _CHUNK), jnp.float32),
            pltpu.SemaphoreType.DMA,
        ],
        compiler_params=pltpu.CompilerParams(use_tc_tiling_on_sc=False),
    )

_BLK = 2048
_NBLK = _B // _BLK


def _mm(a, b):
    return lax.dot_general(a, b, (((1,), (1,)), ((), ())),
                           preferred_element_type=jnp.float32)


def _tc_body(gmf, mu, mi, bs, w1, b1, w2, b2, wf, gb, bfs, out):
    w1v = w1[...]
    wfv = wf[...]
    h = _mm(mu[...], w1v[:, :_D]) + _mm(mi[...], w1v[:, _D:]) + b1[...]
    h = jnp.maximum(h, 0.0)
    h = _mm(h, w2[...]) + b2[...]
    r = _mm(gmf[...], wfv[:, :_D]) + _mm(h, wfv[:, _D:])
    out[...] = bs[...] + r + (gb[0, 0] + bfs[0, 0])


_row_spec = pl.BlockSpec((_BLK, _D), lambda i: (i, 0))
_full = lambda s: pl.BlockSpec(s, lambda i: (0,) * len(s))

_tc_dense = pl.pallas_call(
    _tc_body,
    grid=(_NBLK,),
    in_specs=[
        _row_spec,
        _row_spec,
        _row_spec,
        pl.BlockSpec((_BLK, 1), lambda i: (i, 0)),
        _full((_D, 2 * _D)),
        _full((1, _D)),
        _full((_D, _D)),
        _full((1, _D)),
        _full((1, 2 * _D)),
        _full((1, 1)),
        _full((1, 1)),
    ],
    out_specs=pl.BlockSpec((_BLK, 1), lambda i: (i, 0)),
    out_shape=jax.ShapeDtypeStruct((_B, 1), jnp.float32),
)


def kernel(d0, d1, d2, d3, d4, user_id, item_id, user_bias, item_bias,
           global_bias, gmf_user_emb, gmf_item_emb, mlp_user_emb, mlp_item_emb,
           W1, b1, W2, b2, Wf, bf):
    gmf, mu_rows, mi_rows, bsum = _make_sc_gather()(
        user_id, item_id, user_bias, item_bias,
        gmf_user_emb, gmf_item_emb, mlp_user_emb, mlp_item_emb)
    out = _tc_dense(
        gmf, mu_rows, mi_rows, bsum.reshape(_B, 1),
        W1, b1.reshape(1, _D), W2, b2.reshape(1, _D), Wf,
        global_bias.reshape(1, 1), bf.reshape(1, 1))
    return out[:, 0]
